# baseline (speedup 1.0000x reference)
import functools

import jax
import jax.numpy as jnp
from jax import lax
from jax.experimental import pallas as pl
from jax.experimental.pallas import tpu as pltpu
from jax.experimental.pallas import tpu_sc as plsc

N_GOS = 100000
D = 128
B = 16384
MARGIN_ = 0.1
EPS = 1e-5

NC = 2
NS = 16
NW = NC * NS
PAIRS_W = B // NW
CH = 128
NCH = PAIRS_W // CH
NCHT = 2 * NCH
LANES = 16
NBUF = 4


def _sc_gather_body(emb, idx_all, c_out, d_out, stats_out,
                    idx_v, rows_v, stats_v, sems):
    wid = lax.axis_index("s") * NC + lax.axis_index("c")
    base = wid * PAIRS_W
    pltpu.sync_copy(idx_all.at[wid], idx_v)

    zeros = [jnp.zeros((LANES,), jnp.float32) for _ in range(2 * (D // LANES))]

    def chunk_stats(rv, acc):
        def row(r, acc):
            sums = list(acc[:D // LANES])
            sqs = list(acc[D // LANES:])
            for k in range(D // LANES):
                v = rv[r, pl.ds(k * LANES, LANES)]
                sums[k] = sums[k] + v
                sqs[k] = sqs[k] + v * v
            return tuple(sums + sqs)
        return lax.fori_loop(0, CH, row, tuple(acc), unroll=4)

    outs = (c_out, d_out)
    gat = [None] * NCHT
    wrb = [None] * NCHT
    acc_c = list(zeros)
    acc_d = list(zeros)
    for j in range(min(NBUF - 1, NCHT)):
        gat[j] = pltpu.async_copy(emb.at[idx_v.at[j]], rows_v.at[j],
                                  sems.at[j])
    for j in range(NCHT):
        b = j % NBUF
        gat[j].wait()
        if j < NCH:
            acc_c = list(chunk_stats(rows_v.at[b], acc_c))
        else:
            acc_d = list(chunk_stats(rows_v.at[b], acc_d))
        dst = outs[j // NCH].at[pl.ds(base + (j % NCH) * CH, CH)]
        wrb[j] = pltpu.async_copy(rows_v.at[b], dst, sems.at[NBUF + b])
        nxt = j + NBUF - 1
        if nxt < NCHT:
            if j >= 1:
                wrb[j - 1].wait()
            gat[nxt] = pltpu.async_copy(emb.at[idx_v.at[nxt]],
                                        rows_v.at[nxt % NBUF],
                                        sems.at[nxt % NBUF])

    for k in range(D // LANES):
        stats_v[0, pl.ds(k * LANES, LANES)] = acc_c[k]
        stats_v[1, pl.ds(k * LANES, LANES)] = acc_c[D // LANES + k]
        stats_v[2, pl.ds(k * LANES, LANES)] = acc_d[k]
        stats_v[3, pl.ds(k * LANES, LANES)] = acc_d[D // LANES + k]
    pltpu.sync_copy(stats_v, stats_out.at[wid])
    for j in range(max(1, NCHT - NBUF + 1), NCHT):
        wrb[j].wait()


@functools.lru_cache(maxsize=1)
def _make_sc_gather():
    mesh = plsc.VectorSubcoreMesh(core_axis_name="c", subcore_axis_name="s")
    return pl.kernel(
        _sc_gather_body,
        out_type=(
            jax.ShapeDtypeStruct((B, D), jnp.float32),
            jax.ShapeDtypeStruct((B, D), jnp.float32),
            jax.ShapeDtypeStruct((NW, 4, D), jnp.float32),
        ),
        mesh=mesh,
        scratch_types=[
            pltpu.VMEM((NCHT, CH), jnp.int32),
            pltpu.VMEM((NBUF, CH, D), jnp.float32),
            pltpu.VMEM((4, D), jnp.float32),
            pltpu.SemaphoreType.DMA((2 * NBUF,)),
        ],
    )


def _sc_rad_body(rad, idx_all, radd_out, idx_v, rad_v, radd_v, sem):
    wid = lax.axis_index("s") * NC + lax.axis_index("c")
    pltpu.sync_copy(idx_all.at[wid], idx_v)
    rad_dmas = []
    for j in range(NCHT):
        rad_dmas.append(
            pltpu.async_copy(rad.at[idx_v.at[j]], rad_v.at[j], sem))
    for dma in rad_dmas:
        dma.wait()
    for j in range(NCH):
        for k in range(CH // LANES):
            sl = pl.ds(k * LANES, LANES)
            radd_v[j, sl] = jnp.abs(rad_v[j, sl]) - jnp.abs(rad_v[NCH + j, sl])
    pltpu.sync_copy(radd_v, radd_out.at[wid])


@functools.lru_cache(maxsize=1)
def _make_sc_rad():
    mesh = plsc.VectorSubcoreMesh(core_axis_name="c", subcore_axis_name="s")
    return pl.kernel(
        _sc_rad_body,
        out_type=jax.ShapeDtypeStruct((NW, NCH, CH), jnp.float32),
        mesh=mesh,
        scratch_types=[
            pltpu.VMEM((NCHT, CH), jnp.int32),
            pltpu.VMEM((NCHT, CH), jnp.float32),
            pltpu.VMEM((NCH, CH), jnp.float32),
            pltpu.SemaphoreType.DMA,
        ],
    )


NB = 4
BR = B // NB


def _tc_body(c_ref, d_ref, radd_ref, stats_ref, g_ref, out_ref,
             coef_v, radd_m, acc_v):
    j = pl.program_id(0)

    @pl.when(j == 0)
    def _prep():
        n = jnp.float32(B)
        g = g_ref[...]
        sum_c = jnp.sum(stats_ref[:, 0:1, :], axis=0)
        sq_c = jnp.sum(stats_ref[:, 1:2, :], axis=0)
        sum_d = jnp.sum(stats_ref[:, 2:3, :], axis=0)
        sq_d = jnp.sum(stats_ref[:, 3:4, :], axis=0)
        mu_c = sum_c / n
        var_c = sq_c / n - mu_c * mu_c
        mu_d = sum_d / n
        var_d = sq_d / n - mu_d * mu_d
        inv_c = g / jnp.sqrt(var_c + EPS)
        inv_d = g / jnp.sqrt(var_d + EPS)
        coef_v[0:1, :] = inv_c
        coef_v[1:2, :] = inv_d
        coef_v[2:3, :] = mu_d * inv_d - mu_c * inv_c
        radd_m[...] = radd_ref[...].reshape(B // D, D)
        acc_v[0, 0] = jnp.float32(0.0)

    a = coef_v[0:1, :]
    bb = coef_v[1:2, :]
    off = coef_v[2:3, :]
    x = c_ref[...] * a - d_ref[...] * bb + off
    s = jnp.sum(x * x, axis=1)
    sm = s.reshape(BR // D, D)
    ra = radd_m[pl.ds(j * (BR // D), BR // D), :]
    dist = jnp.sqrt(sm) + ra - MARGIN_
    acc_v[0, 0] += jnp.sum(jnp.maximum(dist, 0.0))

    @pl.when(j == NB - 1)
    def _emit():
        out_ref[...] = jnp.full((1, 1), acc_v[0, 0] / jnp.float32(B), jnp.float32)


_tc_finalize = pl.pallas_call(
    _tc_body,
    grid=(NB,),
    in_specs=[
        pl.BlockSpec((BR, D), lambda j: (j, 0)),
        pl.BlockSpec((BR, D), lambda j: (j, 0)),
        pl.BlockSpec((NW, NCH, CH), lambda j: (0, 0, 0)),
        pl.BlockSpec((NW, 4, D), lambda j: (0, 0, 0)),
        pl.BlockSpec((1, D), lambda j: (0, 0)),
    ],
    out_specs=pl.BlockSpec((1, 1), lambda j: (0, 0)),
    out_shape=jax.ShapeDtypeStruct((1, 1), jnp.float32),
    scratch_shapes=[
        pltpu.VMEM((4, D), jnp.float32),
        pltpu.VMEM((B // D, D), jnp.float32),
        pltpu.SMEM((1, 1), jnp.float32),
    ],
)


def kernel(data, go_embed_weight, go_rad_weight, bn_weight, bn_bias):
    del bn_bias
    idx_all = (data.reshape(NW, NCH, CH, 2)
               .transpose(0, 3, 1, 2)
               .reshape(NW, NCHT, CH))
    rad1 = go_rad_weight.reshape(N_GOS)
    c_raw, d_raw, stats = _make_sc_gather()(go_embed_weight, idx_all)
    radd = _make_sc_rad()(rad1, idx_all)
    loss = _tc_finalize(c_raw, d_raw, radd, stats, bn_weight.reshape(1, D))
    return loss[0, 0]


# speedup vs baseline: 1.1362x; 1.1362x over previous
import functools

import jax
import jax.numpy as jnp
from jax import lax
from jax.experimental import pallas as pl
from jax.experimental.pallas import tpu as pltpu
from jax.experimental.pallas import tpu_sc as plsc

N_GOS = 100000
D = 128
B = 16384
MARGIN_ = 0.1
EPS = 1e-5

NC = 2
NS = 16
NW = NC * NS
PAIRS_W = B // NW
CH = 128
NCH = PAIRS_W // CH
NCHT = 2 * NCH
LANES = 16
NBUF = 4


def _sc_gather_body(emb, rad, idx_all, c_out, d_out, radd_out, stats_out,
                    idx_v, rows_v, rad_v, radd_v, stats_v, sems):
    wid = lax.axis_index("s") * NC + lax.axis_index("c")
    base = wid * PAIRS_W
    pltpu.sync_copy(idx_all.at[wid], idx_v)

    rad_dmas = []
    for j in range(NCHT):
        rad_dmas.append(
            pltpu.async_copy(rad.at[idx_v.at[j]], rad_v.at[j],
                             sems.at[2 * NBUF]))

    zeros = [jnp.zeros((LANES,), jnp.float32) for _ in range(2 * (D // LANES))]

    def chunk_stats(rv, acc):
        def row(r, acc):
            sums = list(acc[:D // LANES])
            sqs = list(acc[D // LANES:])
            for k in range(D // LANES):
                v = rv[r, pl.ds(k * LANES, LANES)]
                sums[k] = sums[k] + v
                sqs[k] = sqs[k] + v * v
            return tuple(sums + sqs)
        return lax.fori_loop(0, CH, row, tuple(acc), unroll=4)

    outs = (c_out, d_out)
    gat = [None] * NCHT
    wrb = [None] * NCHT
    acc_c = list(zeros)
    acc_d = list(zeros)
    for j in range(min(NBUF - 1, NCHT)):
        gat[j] = pltpu.async_copy(emb.at[idx_v.at[j]], rows_v.at[j],
                                  sems.at[j])
    for j in range(NCHT):
        b = j % NBUF
        gat[j].wait()
        if j < NCH:
            acc_c = list(chunk_stats(rows_v.at[b], acc_c))
        else:
            acc_d = list(chunk_stats(rows_v.at[b], acc_d))
        dst = outs[j // NCH].at[pl.ds(base + (j % NCH) * CH, CH)]
        wrb[j] = pltpu.async_copy(rows_v.at[b], dst, sems.at[NBUF + b])
        nxt = j + NBUF - 1
        if nxt < NCHT:
            if j >= 1:
                wrb[j - 1].wait()
            gat[nxt] = pltpu.async_copy(emb.at[idx_v.at[nxt]],
                                        rows_v.at[nxt % NBUF],
                                        sems.at[nxt % NBUF])

    for k in range(D // LANES):
        stats_v[0, pl.ds(k * LANES, LANES)] = acc_c[k]
        stats_v[1, pl.ds(k * LANES, LANES)] = acc_c[D // LANES + k]
        stats_v[2, pl.ds(k * LANES, LANES)] = acc_d[k]
        stats_v[3, pl.ds(k * LANES, LANES)] = acc_d[D // LANES + k]
    pltpu.sync_copy(stats_v, stats_out.at[wid])

    for dma in rad_dmas:
        dma.wait()
    for j in range(NCH):
        for k in range(CH // LANES):
            sl = pl.ds(k * LANES, LANES)
            radd_v[j, sl] = jnp.abs(rad_v[j, sl]) - jnp.abs(rad_v[NCH + j, sl])
    pltpu.sync_copy(radd_v, radd_out.at[wid])
    for j in range(max(1, NCHT - NBUF + 1), NCHT):
        wrb[j].wait()


@functools.lru_cache(maxsize=1)
def _make_sc_gather():
    mesh = plsc.VectorSubcoreMesh(core_axis_name="c", subcore_axis_name="s")
    return pl.kernel(
        _sc_gather_body,
        out_type=(
            jax.ShapeDtypeStruct((B, D), jnp.float32),
            jax.ShapeDtypeStruct((B, D), jnp.float32),
            jax.ShapeDtypeStruct((NW, NCH, CH), jnp.float32),
            jax.ShapeDtypeStruct((NW, 4, D), jnp.float32),
        ),
        mesh=mesh,
        scratch_types=[
            pltpu.VMEM((NCHT, CH), jnp.int32),
            pltpu.VMEM((NBUF, CH, D), jnp.float32),
            pltpu.VMEM((NCHT, CH), jnp.float32),
            pltpu.VMEM((NCH, CH), jnp.float32),
            pltpu.VMEM((4, D), jnp.float32),
            pltpu.SemaphoreType.DMA((2 * NBUF + 1,)),
        ],
    )


NB = 4
BR = B // NB


def _tc_body(c_ref, d_ref, radd_ref, stats_ref, g_ref, out_ref,
             coef_v, radd_m, acc_v):
    j = pl.program_id(0)

    @pl.when(j == 0)
    def _prep():
        n = jnp.float32(B)
        g = g_ref[...]
        sum_c = jnp.sum(stats_ref[:, 0:1, :], axis=0)
        sq_c = jnp.sum(stats_ref[:, 1:2, :], axis=0)
        sum_d = jnp.sum(stats_ref[:, 2:3, :], axis=0)
        sq_d = jnp.sum(stats_ref[:, 3:4, :], axis=0)
        mu_c = sum_c / n
        var_c = sq_c / n - mu_c * mu_c
        mu_d = sum_d / n
        var_d = sq_d / n - mu_d * mu_d
        inv_c = g / jnp.sqrt(var_c + EPS)
        inv_d = g / jnp.sqrt(var_d + EPS)
        coef_v[0:1, :] = inv_c
        coef_v[1:2, :] = inv_d
        coef_v[2:3, :] = mu_d * inv_d - mu_c * inv_c
        radd_m[...] = radd_ref[...].reshape(B // D, D)
        acc_v[0, 0] = jnp.float32(0.0)

    a = coef_v[0:1, :]
    bb = coef_v[1:2, :]
    off = coef_v[2:3, :]
    x = c_ref[...] * a - d_ref[...] * bb + off
    s = jnp.sum(x * x, axis=1)
    sm = s.reshape(BR // D, D)
    ra = radd_m[pl.ds(j * (BR // D), BR // D), :]
    dist = jnp.sqrt(sm) + ra - MARGIN_
    acc_v[0, 0] += jnp.sum(jnp.maximum(dist, 0.0))

    @pl.when(j == NB - 1)
    def _emit():
        out_ref[...] = jnp.full((1, 1), acc_v[0, 0] / jnp.float32(B), jnp.float32)


_tc_finalize = pl.pallas_call(
    _tc_body,
    grid=(NB,),
    in_specs=[
        pl.BlockSpec((BR, D), lambda j: (j, 0)),
        pl.BlockSpec((BR, D), lambda j: (j, 0)),
        pl.BlockSpec((NW, NCH, CH), lambda j: (0, 0, 0)),
        pl.BlockSpec((NW, 4, D), lambda j: (0, 0, 0)),
        pl.BlockSpec((1, D), lambda j: (0, 0)),
    ],
    out_specs=pl.BlockSpec((1, 1), lambda j: (0, 0)),
    out_shape=jax.ShapeDtypeStruct((1, 1), jnp.float32),
    scratch_shapes=[
        pltpu.VMEM((4, D), jnp.float32),
        pltpu.VMEM((B // D, D), jnp.float32),
        pltpu.SMEM((1, 1), jnp.float32),
    ],
)


def kernel(data, go_embed_weight, go_rad_weight, bn_weight, bn_bias):
    del bn_bias
    idx_all = (data.reshape(NW, NCH, CH, 2)
               .transpose(0, 3, 1, 2)
               .reshape(NW, NCHT, CH))
    rad1 = go_rad_weight.reshape(N_GOS)
    c_raw, d_raw, radd, stats = _make_sc_gather()(
        go_embed_weight, rad1, idx_all)
    loss = _tc_finalize(c_raw, d_raw, radd, stats, bn_weight.reshape(1, D))
    return loss[0, 0]


# speedup vs baseline: 1.1710x; 1.0306x over previous
import functools

import jax
import jax.numpy as jnp
from jax import lax
from jax.experimental import pallas as pl
from jax.experimental.pallas import tpu as pltpu
from jax.experimental.pallas import tpu_sc as plsc

N_GOS = 100000
D = 128
B = 16384
MARGIN_ = 0.1
EPS = 1e-5

NC = 2
NS = 16
NW = NC * NS
PAIRS_W = B // NW
CH = 128
NCH = PAIRS_W // CH
NCHT = 2 * NCH
LANES = 16
NBUF = 6


def _sc_gather_body(emb, rad, idx_all, c_out, d_out, radd_out, stats_out,
                    idx_v, rows_v, rad_v, radd_v, stats_v, sems):
    wid = lax.axis_index("s") * NC + lax.axis_index("c")
    base = wid * PAIRS_W
    pltpu.sync_copy(idx_all.at[wid], idx_v)

    rad_dmas = []
    for j in range(NCHT):
        rad_dmas.append(
            pltpu.async_copy(rad.at[idx_v.at[j]], rad_v.at[j],
                             sems.at[2 * NBUF]))

    zeros = [jnp.zeros((LANES,), jnp.float32) for _ in range(2 * (D // LANES))]

    def chunk_stats(rv, acc):
        def row(r, acc):
            sums = list(acc[:D // LANES])
            sqs = list(acc[D // LANES:])
            for k in range(D // LANES):
                v = rv[r, pl.ds(k * LANES, LANES)]
                sums[k] = sums[k] + v
                sqs[k] = sqs[k] + v * v
            return tuple(sums + sqs)
        return lax.fori_loop(0, CH, row, tuple(acc), unroll=2)

    outs = (c_out, d_out)
    gat = [None] * NCHT
    wrb = [None] * NCHT
    acc_c = list(zeros)
    acc_d = list(zeros)
    for j in range(min(NBUF - 1, NCHT)):
        gat[j] = pltpu.async_copy(emb.at[idx_v.at[j]], rows_v.at[j],
                                  sems.at[j])
    for j in range(NCHT):
        b = j % NBUF
        gat[j].wait()
        if j < NCH:
            acc_c = list(chunk_stats(rows_v.at[b], acc_c))
        else:
            acc_d = list(chunk_stats(rows_v.at[b], acc_d))
        dst = outs[j // NCH].at[pl.ds(base + (j % NCH) * CH, CH)]
        wrb[j] = pltpu.async_copy(rows_v.at[b], dst, sems.at[NBUF + b])
        nxt = j + NBUF - 1
        if nxt < NCHT:
            if j >= 1:
                wrb[j - 1].wait()
            gat[nxt] = pltpu.async_copy(emb.at[idx_v.at[nxt]],
                                        rows_v.at[nxt % NBUF],
                                        sems.at[nxt % NBUF])

    for k in range(D // LANES):
        stats_v[0, pl.ds(k * LANES, LANES)] = acc_c[k]
        stats_v[1, pl.ds(k * LANES, LANES)] = acc_c[D // LANES + k]
        stats_v[2, pl.ds(k * LANES, LANES)] = acc_d[k]
        stats_v[3, pl.ds(k * LANES, LANES)] = acc_d[D // LANES + k]
    pltpu.sync_copy(stats_v, stats_out.at[wid])

    for dma in rad_dmas:
        dma.wait()
    for j in range(NCH):
        for k in range(CH // LANES):
            sl = pl.ds(k * LANES, LANES)
            radd_v[j, sl] = jnp.abs(rad_v[j, sl]) - jnp.abs(rad_v[NCH + j, sl])
    pltpu.sync_copy(radd_v, radd_out.at[wid])
    for j in range(max(1, NCHT - NBUF + 1), NCHT):
        wrb[j].wait()


@functools.lru_cache(maxsize=1)
def _make_sc_gather():
    mesh = plsc.VectorSubcoreMesh(core_axis_name="c", subcore_axis_name="s")
    return pl.kernel(
        _sc_gather_body,
        out_type=(
            jax.ShapeDtypeStruct((B, D), jnp.float32),
            jax.ShapeDtypeStruct((B, D), jnp.float32),
            jax.ShapeDtypeStruct((NW, NCH, CH), jnp.float32),
            jax.ShapeDtypeStruct((NW, 4, D), jnp.float32),
        ),
        mesh=mesh,
        scratch_types=[
            pltpu.VMEM((NCHT, CH), jnp.int32),
            pltpu.VMEM((NBUF, CH, D), jnp.float32),
            pltpu.VMEM((NCHT, CH), jnp.float32),
            pltpu.VMEM((NCH, CH), jnp.float32),
            pltpu.VMEM((4, D), jnp.float32),
            pltpu.SemaphoreType.DMA((2 * NBUF + 1,)),
        ],
    )


NB = 2
BR = B // NB


def _tc_body(c_ref, d_ref, radd_ref, stats_ref, g_ref, out_ref,
             coef_v, radd_m, acc_v):
    j = pl.program_id(0)

    @pl.when(j == 0)
    def _prep():
        n = jnp.float32(B)
        g = g_ref[...]
        sum_c = jnp.sum(stats_ref[:, 0:1, :], axis=0)
        sq_c = jnp.sum(stats_ref[:, 1:2, :], axis=0)
        sum_d = jnp.sum(stats_ref[:, 2:3, :], axis=0)
        sq_d = jnp.sum(stats_ref[:, 3:4, :], axis=0)
        mu_c = sum_c / n
        var_c = sq_c / n - mu_c * mu_c
        mu_d = sum_d / n
        var_d = sq_d / n - mu_d * mu_d
        inv_c = g / jnp.sqrt(var_c + EPS)
        inv_d = g / jnp.sqrt(var_d + EPS)
        coef_v[0:1, :] = inv_c
        coef_v[1:2, :] = inv_d
        coef_v[2:3, :] = mu_d * inv_d - mu_c * inv_c
        radd_m[...] = radd_ref[...].reshape(B // D, D)
        acc_v[0, 0] = jnp.float32(0.0)

    a = coef_v[0:1, :]
    bb = coef_v[1:2, :]
    off = coef_v[2:3, :]
    x = c_ref[...] * a - d_ref[...] * bb + off
    s = jnp.sum(x * x, axis=1)
    sm = s.reshape(BR // D, D)
    ra = radd_m[pl.ds(j * (BR // D), BR // D), :]
    dist = jnp.sqrt(sm) + ra - MARGIN_
    acc_v[0, 0] += jnp.sum(jnp.maximum(dist, 0.0))

    @pl.when(j == NB - 1)
    def _emit():
        out_ref[...] = jnp.full((1, 1), acc_v[0, 0] / jnp.float32(B), jnp.float32)


_tc_finalize = pl.pallas_call(
    _tc_body,
    grid=(NB,),
    in_specs=[
        pl.BlockSpec((BR, D), lambda j: (j, 0)),
        pl.BlockSpec((BR, D), lambda j: (j, 0)),
        pl.BlockSpec((NW, NCH, CH), lambda j: (0, 0, 0)),
        pl.BlockSpec((NW, 4, D), lambda j: (0, 0, 0)),
        pl.BlockSpec((1, D), lambda j: (0, 0)),
    ],
    out_specs=pl.BlockSpec((1, 1), lambda j: (0, 0)),
    out_shape=jax.ShapeDtypeStruct((1, 1), jnp.float32),
    scratch_shapes=[
        pltpu.VMEM((4, D), jnp.float32),
        pltpu.VMEM((B // D, D), jnp.float32),
        pltpu.SMEM((1, 1), jnp.float32),
    ],
)


def kernel(data, go_embed_weight, go_rad_weight, bn_weight, bn_bias):
    del bn_bias
    idx_all = (data.reshape(NW, NCH, CH, 2)
               .transpose(0, 3, 1, 2)
               .reshape(NW, NCHT, CH))
    rad1 = go_rad_weight.reshape(N_GOS)
    c_raw, d_raw, radd, stats = _make_sc_gather()(
        go_embed_weight, rad1, idx_all)
    loss = _tc_finalize(c_raw, d_raw, radd, stats, bn_weight.reshape(1, D))
    return loss[0, 0]


# speedup vs baseline: 1.1785x; 1.0064x over previous
import functools

import jax
import jax.numpy as jnp
from jax import lax
from jax.experimental import pallas as pl
from jax.experimental.pallas import tpu as pltpu
from jax.experimental.pallas import tpu_sc as plsc

N_GOS = 100000
D = 128
B = 16384
MARGIN_ = 0.1
EPS = 1e-5

NC = 2
NS = 16
NW = NC * NS
PAIRS_W = B // NW
CH = 128
NCH = PAIRS_W // CH
NCHT = 2 * NCH
LANES = 16
NBUF = 6


def _sc_gather_body(emb, rad, idx_all, c_out, d_out, radd_out, stats_out,
                    idx_v, rows_v, rad_v, radd_v, stats_v, sems):
    wid = lax.axis_index("s") * NC + lax.axis_index("c")
    base = wid * PAIRS_W
    pltpu.sync_copy(idx_all.at[wid], idx_v)

    rad_dmas = []
    for j in range(NCHT):
        rad_dmas.append(
            pltpu.async_copy(rad.at[idx_v.at[j]], rad_v.at[j],
                             sems.at[2 * NBUF]))

    zeros = [jnp.zeros((LANES,), jnp.float32) for _ in range(2 * (D // LANES))]

    def chunk_stats(rv, acc):
        def row(r, acc):
            sums = list(acc[:D // LANES])
            sqs = list(acc[D // LANES:])
            for k in range(D // LANES):
                v = rv[r, pl.ds(k * LANES, LANES)]
                sums[k] = sums[k] + v
                sqs[k] = sqs[k] + v * v
            return tuple(sums + sqs)
        return lax.fori_loop(0, CH, row, tuple(acc), unroll=4)

    outs = (c_out, d_out)
    gat = [None] * NCHT
    wrb = [None] * NCHT
    acc_c = list(zeros)
    acc_d = list(zeros)
    for j in range(min(NBUF - 1, NCHT)):
        gat[j] = pltpu.async_copy(emb.at[idx_v.at[j]], rows_v.at[j],
                                  sems.at[j])
    for j in range(NCHT):
        b = j % NBUF
        gat[j].wait()
        if j < NCH:
            acc_c = list(chunk_stats(rows_v.at[b], acc_c))
        else:
            acc_d = list(chunk_stats(rows_v.at[b], acc_d))
        dst = outs[j // NCH].at[pl.ds(base + (j % NCH) * CH, CH)]
        wrb[j] = pltpu.async_copy(rows_v.at[b], dst, sems.at[NBUF + b])
        nxt = j + NBUF - 1
        if nxt < NCHT:
            if j >= 1:
                wrb[j - 1].wait()
            gat[nxt] = pltpu.async_copy(emb.at[idx_v.at[nxt]],
                                        rows_v.at[nxt % NBUF],
                                        sems.at[nxt % NBUF])

    for k in range(D // LANES):
        stats_v[0, pl.ds(k * LANES, LANES)] = acc_c[k]
        stats_v[1, pl.ds(k * LANES, LANES)] = acc_c[D // LANES + k]
        stats_v[2, pl.ds(k * LANES, LANES)] = acc_d[k]
        stats_v[3, pl.ds(k * LANES, LANES)] = acc_d[D // LANES + k]
    pltpu.sync_copy(stats_v, stats_out.at[wid])

    for dma in rad_dmas:
        dma.wait()
    for j in range(NCH):
        for k in range(CH // LANES):
            sl = pl.ds(k * LANES, LANES)
            radd_v[j, sl] = jnp.abs(rad_v[j, sl]) - jnp.abs(rad_v[NCH + j, sl])
    pltpu.sync_copy(radd_v, radd_out.at[wid])
    for j in range(max(1, NCHT - NBUF + 1), NCHT):
        wrb[j].wait()


@functools.lru_cache(maxsize=1)
def _make_sc_gather():
    mesh = plsc.VectorSubcoreMesh(core_axis_name="c", subcore_axis_name="s")
    return pl.kernel(
        _sc_gather_body,
        out_type=(
            jax.ShapeDtypeStruct((B, D), jnp.float32),
            jax.ShapeDtypeStruct((B, D), jnp.float32),
            jax.ShapeDtypeStruct((NW, NCH, CH), jnp.float32),
            jax.ShapeDtypeStruct((NW, 4, D), jnp.float32),
        ),
        mesh=mesh,
        scratch_types=[
            pltpu.VMEM((NCHT, CH), jnp.int32),
            pltpu.VMEM((NBUF, CH, D), jnp.float32),
            pltpu.VMEM((NCHT, CH), jnp.float32),
            pltpu.VMEM((NCH, CH), jnp.float32),
            pltpu.VMEM((4, D), jnp.float32),
            pltpu.SemaphoreType.DMA((2 * NBUF + 1,)),
        ],
    )


NB = 4
BR = B // NB


def _tc_body(c_ref, d_ref, radd_ref, stats_ref, g_ref, out_ref,
             coef_v, radd_m, acc_v):
    j = pl.program_id(0)

    @pl.when(j == 0)
    def _prep():
        n = jnp.float32(B)
        g = g_ref[...]
        sum_c = jnp.sum(stats_ref[:, 0:1, :], axis=0)
        sq_c = jnp.sum(stats_ref[:, 1:2, :], axis=0)
        sum_d = jnp.sum(stats_ref[:, 2:3, :], axis=0)
        sq_d = jnp.sum(stats_ref[:, 3:4, :], axis=0)
        mu_c = sum_c / n
        var_c = sq_c / n - mu_c * mu_c
        mu_d = sum_d / n
        var_d = sq_d / n - mu_d * mu_d
        inv_c = g / jnp.sqrt(var_c + EPS)
        inv_d = g / jnp.sqrt(var_d + EPS)
        coef_v[0:1, :] = inv_c
        coef_v[1:2, :] = inv_d
        coef_v[2:3, :] = mu_d * inv_d - mu_c * inv_c
        radd_m[...] = radd_ref[...].reshape(B // D, D)
        acc_v[0, 0] = jnp.float32(0.0)

    a = coef_v[0:1, :]
    bb = coef_v[1:2, :]
    off = coef_v[2:3, :]
    x = c_ref[...] * a - d_ref[...] * bb + off
    s = jnp.sum(x * x, axis=1)
    sm = s.reshape(BR // D, D)
    ra = radd_m[pl.ds(j * (BR // D), BR // D), :]
    dist = jnp.sqrt(sm) + ra - MARGIN_
    acc_v[0, 0] += jnp.sum(jnp.maximum(dist, 0.0))

    @pl.when(j == NB - 1)
    def _emit():
        out_ref[...] = jnp.full((1, 1), acc_v[0, 0] / jnp.float32(B), jnp.float32)


_tc_finalize = pl.pallas_call(
    _tc_body,
    grid=(NB,),
    in_specs=[
        pl.BlockSpec((BR, D), lambda j: (j, 0)),
        pl.BlockSpec((BR, D), lambda j: (j, 0)),
        pl.BlockSpec((NW, NCH, CH), lambda j: (0, 0, 0)),
        pl.BlockSpec((NW, 4, D), lambda j: (0, 0, 0)),
        pl.BlockSpec((1, D), lambda j: (0, 0)),
    ],
    out_specs=pl.BlockSpec((1, 1), lambda j: (0, 0)),
    out_shape=jax.ShapeDtypeStruct((1, 1), jnp.float32),
    scratch_shapes=[
        pltpu.VMEM((4, D), jnp.float32),
        pltpu.VMEM((B // D, D), jnp.float32),
        pltpu.SMEM((1, 1), jnp.float32),
    ],
)


def kernel(data, go_embed_weight, go_rad_weight, bn_weight, bn_bias):
    del bn_bias
    idx_all = (data.reshape(NW, NCH, CH, 2)
               .transpose(0, 3, 1, 2)
               .reshape(NW, NCHT, CH))
    rad1 = go_rad_weight.reshape(N_GOS)
    c_raw, d_raw, radd, stats = _make_sc_gather()(
        go_embed_weight, rad1, idx_all)
    loss = _tc_finalize(c_raw, d_raw, radd, stats, bn_weight.reshape(1, D))
    return loss[0, 0]


# speedup vs baseline: 1.1888x; 1.0088x over previous
import functools

import jax
import jax.numpy as jnp
from jax import lax
from jax.experimental import pallas as pl
from jax.experimental.pallas import tpu as pltpu
from jax.experimental.pallas import tpu_sc as plsc

N_GOS = 100000
D = 128
B = 16384
MARGIN_ = 0.1
EPS = 1e-5

NC = 2
NS = 16
NW = NC * NS
PAIRS_W = B // NW
CH = 128
NCH = PAIRS_W // CH
NCHT = 2 * NCH
LANES = 16
NBUF = 6


def _sc_gather_body(emb, rad, idx_all, c_out, d_out, radd_out, stats_out,
                    idx_v, rows_v, rad_v, radd_v, stats_v, sems):
    wid = lax.axis_index("s") * NC + lax.axis_index("c")
    base = wid * PAIRS_W
    pltpu.sync_copy(idx_all.at[wid], idx_v)

    rad_dmas = []
    for j in range(NCHT):
        rad_dmas.append(
            pltpu.async_copy(rad.at[idx_v.at[j]], rad_v.at[j],
                             sems.at[2 * NBUF]))

    zeros = [jnp.zeros((LANES,), jnp.float32) for _ in range(2 * (D // LANES))]

    def chunk_stats(rv, acc):
        def row(r, acc):
            sums = list(acc[:D // LANES])
            sqs = list(acc[D // LANES:])
            for k in range(D // LANES):
                v = rv[r, pl.ds(k * LANES, LANES)]
                sums[k] = sums[k] + v
                sqs[k] = sqs[k] + v * v
            return tuple(sums + sqs)
        return lax.fori_loop(0, CH, row, tuple(acc), unroll=2)

    outs = (c_out, d_out)
    gat = [None] * NCHT
    wrb = [None] * NCHT
    acc_c = list(zeros)
    acc_d = list(zeros)
    for j in range(min(NBUF - 1, NCHT)):
        gat[j] = pltpu.async_copy(emb.at[idx_v.at[j]], rows_v.at[j],
                                  sems.at[j])
    for j in range(NCHT):
        b = j % NBUF
        gat[j].wait()
        if j < NCH:
            acc_c = list(chunk_stats(rows_v.at[b], acc_c))
        else:
            acc_d = list(chunk_stats(rows_v.at[b], acc_d))
        dst = outs[j // NCH].at[pl.ds(base + (j % NCH) * CH, CH)]
        wrb[j] = pltpu.async_copy(rows_v.at[b], dst, sems.at[NBUF + b])
        nxt = j + NBUF - 1
        if nxt < NCHT:
            if j >= 1:
                wrb[j - 1].wait()
            gat[nxt] = pltpu.async_copy(emb.at[idx_v.at[nxt]],
                                        rows_v.at[nxt % NBUF],
                                        sems.at[nxt % NBUF])

    for k in range(D // LANES):
        stats_v[0, pl.ds(k * LANES, LANES)] = acc_c[k]
        stats_v[1, pl.ds(k * LANES, LANES)] = acc_c[D // LANES + k]
        stats_v[2, pl.ds(k * LANES, LANES)] = acc_d[k]
        stats_v[3, pl.ds(k * LANES, LANES)] = acc_d[D // LANES + k]
    pltpu.sync_copy(stats_v, stats_out.at[wid])

    for dma in rad_dmas:
        dma.wait()
    for j in range(NCH):
        for k in range(CH // LANES):
            sl = pl.ds(k * LANES, LANES)
            radd_v[j, sl] = jnp.abs(rad_v[j, sl]) - jnp.abs(rad_v[NCH + j, sl])
    pltpu.sync_copy(radd_v, radd_out.at[wid])
    for j in range(max(1, NCHT - NBUF + 1), NCHT):
        wrb[j].wait()


@functools.lru_cache(maxsize=1)
def _make_sc_gather():
    mesh = plsc.VectorSubcoreMesh(core_axis_name="c", subcore_axis_name="s")
    return pl.kernel(
        _sc_gather_body,
        out_type=(
            jax.ShapeDtypeStruct((B, D), jnp.float32),
            jax.ShapeDtypeStruct((B, D), jnp.float32),
            jax.ShapeDtypeStruct((NW, NCH, CH), jnp.float32),
            jax.ShapeDtypeStruct((NW, 4, D), jnp.float32),
        ),
        mesh=mesh,
        scratch_types=[
            pltpu.VMEM((NCHT, CH), jnp.int32),
            pltpu.VMEM((NBUF, CH, D), jnp.float32),
            pltpu.VMEM((NCHT, CH), jnp.float32),
            pltpu.VMEM((NCH, CH), jnp.float32),
            pltpu.VMEM((4, D), jnp.float32),
            pltpu.SemaphoreType.DMA((2 * NBUF + 1,)),
        ],
    )


NB = 4
BR = B // NB


def _tc_body(c_ref, d_ref, radd_ref, stats_ref, g_ref, out_ref,
             coef_v, radd_m, acc_v):
    j = pl.program_id(0)

    @pl.when(j == 0)
    def _prep():
        n = jnp.float32(B)
        g = g_ref[...]
        sum_c = jnp.sum(stats_ref[:, 0:1, :], axis=0)
        sq_c = jnp.sum(stats_ref[:, 1:2, :], axis=0)
        sum_d = jnp.sum(stats_ref[:, 2:3, :], axis=0)
        sq_d = jnp.sum(stats_ref[:, 3:4, :], axis=0)
        mu_c = sum_c / n
        var_c = sq_c / n - mu_c * mu_c
        mu_d = sum_d / n
        var_d = sq_d / n - mu_d * mu_d
        inv_c = g / jnp.sqrt(var_c + EPS)
        inv_d = g / jnp.sqrt(var_d + EPS)
        coef_v[0:1, :] = inv_c
        coef_v[1:2, :] = inv_d
        coef_v[2:3, :] = mu_d * inv_d - mu_c * inv_c
        radd_m[...] = radd_ref[...].reshape(B // D, D)
        acc_v[0, 0] = jnp.float32(0.0)

    a = coef_v[0:1, :]
    bb = coef_v[1:2, :]
    off = coef_v[2:3, :]
    x = c_ref[...] * a - d_ref[...] * bb + off
    s = jnp.sum(x * x, axis=1)
    sm = s.reshape(BR // D, D)
    ra = radd_m[pl.ds(j * (BR // D), BR // D), :]
    dist = jnp.sqrt(sm) + ra - MARGIN_
    acc_v[0, 0] += jnp.sum(jnp.maximum(dist, 0.0))

    @pl.when(j == NB - 1)
    def _emit():
        out_ref[...] = jnp.full((1, 1), acc_v[0, 0] / jnp.float32(B), jnp.float32)


_tc_finalize = pl.pallas_call(
    _tc_body,
    grid=(NB,),
    in_specs=[
        pl.BlockSpec((BR, D), lambda j: (j, 0)),
        pl.BlockSpec((BR, D), lambda j: (j, 0)),
        pl.BlockSpec((NW, NCH, CH), lambda j: (0, 0, 0)),
        pl.BlockSpec((NW, 4, D), lambda j: (0, 0, 0)),
        pl.BlockSpec((1, D), lambda j: (0, 0)),
    ],
    out_specs=pl.BlockSpec((1, 1), lambda j: (0, 0)),
    out_shape=jax.ShapeDtypeStruct((1, 1), jnp.float32),
    scratch_shapes=[
        pltpu.VMEM((4, D), jnp.float32),
        pltpu.VMEM((B // D, D), jnp.float32),
        pltpu.SMEM((1, 1), jnp.float32),
    ],
)


def kernel(data, go_embed_weight, go_rad_weight, bn_weight, bn_bias):
    del bn_bias
    idx_all = (data.reshape(NW, NCH, CH, 2)
               .transpose(0, 3, 1, 2)
               .reshape(NW, NCHT, CH))
    rad1 = go_rad_weight.reshape(N_GOS)
    c_raw, d_raw, radd, stats = _make_sc_gather()(
        go_embed_weight, rad1, idx_all)
    loss = _tc_finalize(c_raw, d_raw, radd, stats, bn_weight.reshape(1, D))
    return loss[0, 0]
